# Initial kernel scaffold; baseline (speedup 1.0000x reference)
import jax
import jax.numpy as jnp
from jax.experimental import pallas as pl


def kernel(x, edge_index, W1a, b1a, W1b, b1b, W2a, b2a, W2b, b2b, Wm1, bm1, Wm2, bm2):
    raise NotImplementedError("write your pallas kernel here")


# speedup vs baseline: 3.0823x; 3.0823x over previous
import jax
import jax.numpy as jnp
from jax import lax
from jax.experimental import pallas as pl
from jax.experimental.pallas import tpu as pltpu
from jax.experimental.pallas import tpu_sc as plsc

N = 10000
NPAD = 10240
E = 320000
EPAD = 327680
PADE = EPAD - E
DIN = 128
DH = 256
HALF = 128
NC = 2
NS = 16
ROWS_PER_SUB = NPAD // NS
CHUNK = 128
NCH1 = EPAD // (NC * NS * CHUNK)
NCH2 = EPAD // (NS * CHUNK)
IBLK = 16
RB = 128
BR = 512

_MESH = plsc.VectorSubcoreMesh(core_axis_name="c", subcore_axis_name="s")


def _zero_rows(rows):
    @pl.loop(0, RB)
    def _(i):
        for j in range(HALF // 16):
            rows[i, pl.ds(j * 16, 16)] = jnp.zeros((16,), jnp.float32)


def _agg1_body(x_hbm, src_hbm, dst_hbm, out_hbm, sidx, didx, rows, accum, sem):
    c = lax.axis_index("c")
    s = lax.axis_index("s")
    _zero_rows(rows)

    @pl.loop(0, ROWS_PER_SUB // RB)
    def _(k):
        pltpu.sync_copy(rows, accum.at[pl.ds(s * ROWS_PER_SUB + k * RB, RB)])

    plsc.subcore_barrier()

    @pl.loop(0, NCH1 // IBLK)
    def _(b):
        pltpu.sync_copy(src_hbm.at[c, s, pl.ds(b * IBLK, IBLK)], sidx)
        pltpu.sync_copy(dst_hbm.at[c, s, pl.ds(b * IBLK, IBLK)], didx)

        @pl.loop(0, IBLK)
        def _(j):
            pltpu.async_copy(x_hbm.at[sidx.at[j]], rows, sem).wait()
            pltpu.sync_copy(rows, accum.at[didx.at[j]], add=True)

    plsc.subcore_barrier()

    @pl.loop(0, ROWS_PER_SUB // RB)
    def _(k):
        base = s * ROWS_PER_SUB + k * RB
        pltpu.sync_copy(accum.at[pl.ds(base, RB)], rows)
        pltpu.sync_copy(rows, out_hbm.at[c, pl.ds(base, RB)])


def _agg2_body(h_hbm, src_hbm, dst_hbm, out_hbm, sidx, didx, rows, accum, sem):
    c = lax.axis_index("c")
    s = lax.axis_index("s")

    @pl.loop(0, ROWS_PER_SUB // RB)
    def _(k):
        base = s * ROWS_PER_SUB + k * RB
        pltpu.sync_copy(h_hbm.at[pl.ds(c * NPAD + base, RB)], rows)
        pltpu.sync_copy(rows, accum.at[pl.ds(base, RB)])

    plsc.subcore_barrier()

    @pl.loop(0, NCH2 // IBLK)
    def _(b):
        pltpu.sync_copy(src_hbm.at[c, s, pl.ds(b * IBLK, IBLK)], sidx)
        pltpu.sync_copy(dst_hbm.at[s, pl.ds(b * IBLK, IBLK)], didx)

        @pl.loop(0, IBLK)
        def _(j):
            pltpu.async_copy(h_hbm.at[sidx.at[j]], rows, sem).wait()
            pltpu.sync_copy(rows, accum.at[didx.at[j]], add=True)

    plsc.subcore_barrier()

    @pl.loop(0, ROWS_PER_SUB // RB)
    def _(k):
        base = s * ROWS_PER_SUB + k * RB
        pltpu.sync_copy(accum.at[pl.ds(base, RB)], rows)
        pltpu.sync_copy(rows, out_hbm.at[c, pl.ds(base, RB)])


_agg1 = pl.kernel(
    _agg1_body,
    out_type=jax.ShapeDtypeStruct((NC, NPAD, DIN), jnp.float32),
    mesh=_MESH,
    scratch_types=[
        pltpu.VMEM((IBLK, CHUNK), jnp.int32),
        pltpu.VMEM((IBLK, CHUNK), jnp.int32),
        pltpu.VMEM((CHUNK, DIN), jnp.float32),
        pltpu.VMEM_SHARED((NPAD, DIN), jnp.float32),
        pltpu.SemaphoreType.DMA,
    ],
)

_agg2 = pl.kernel(
    _agg2_body,
    out_type=jax.ShapeDtypeStruct((NC, NPAD, HALF), jnp.float32),
    mesh=_MESH,
    scratch_types=[
        pltpu.VMEM((IBLK, CHUNK), jnp.int32),
        pltpu.VMEM((IBLK, CHUNK), jnp.int32),
        pltpu.VMEM((CHUNK, HALF), jnp.float32),
        pltpu.VMEM_SHARED((NPAD, HALF), jnp.float32),
        pltpu.SemaphoreType.DMA,
    ],
)


def _mlp1_body(x_ref, a_ref, wa_ref, ba_ref, wb_ref, bb_ref, o_ref):
    z = x_ref[...] + a_ref[0] + a_ref[1]
    t = jnp.dot(z, wa_ref[...], preferred_element_type=jnp.float32) + ba_ref[...]
    t = jnp.maximum(t, 0.0)
    u = jnp.dot(t, wb_ref[...], preferred_element_type=jnp.float32) + bb_ref[...]
    u = jnp.maximum(u, 0.0)
    o_ref[0] = u[:, :HALF]
    o_ref[1] = u[:, HALF:]


def _mlp2_body(z_ref, w2a_ref, b2a_ref, w2b_ref, b2b_ref,
               wm1_ref, bm1_ref, wm2_ref, bm2_ref, o_ref):
    t = (jnp.dot(z_ref[0], w2a_ref[:HALF], preferred_element_type=jnp.float32)
         + jnp.dot(z_ref[1], w2a_ref[HALF:], preferred_element_type=jnp.float32)
         + b2a_ref[...])
    t = jnp.maximum(t, 0.0)
    v = jnp.dot(t, w2b_ref[...], preferred_element_type=jnp.float32) + b2b_ref[...]
    h2 = jnp.maximum(v, 0.0)
    m = jnp.dot(h2, wm1_ref[...], preferred_element_type=jnp.float32) + bm1_ref[...]
    m = jnp.maximum(m, 0.0)
    o_ref[...] = jnp.dot(m, wm2_ref[...], preferred_element_type=jnp.float32) + bm2_ref[...]


def kernel(x, edge_index, W1a, b1a, W1b, b1b, W2a, b2a, W2b, b2b, Wm1, bm1, Wm2, bm2):
    src = edge_index[0].astype(jnp.int32)
    dst = edge_index[1].astype(jnp.int32)
    src_p = jnp.concatenate([src, jnp.zeros((PADE,), jnp.int32)])
    dst_p = jnp.concatenate([dst, jnp.full((PADE,), NPAD - 1, jnp.int32)])

    src1 = src_p.reshape(NC, NS, NCH1, CHUNK)
    dst1 = dst_p.reshape(NC, NS, NCH1, CHUNK)
    src2 = (src_p[None, :] + (jnp.arange(NC, dtype=jnp.int32) * NPAD)[:, None]
            ).reshape(NC, NS, NCH2, CHUNK)
    dst2 = dst_p.reshape(NS, NCH2, CHUNK)

    x_pad = jnp.pad(x, ((0, NPAD - N), (0, 0)))

    a = _agg1(x, src1, dst1)

    h1 = pl.pallas_call(
        _mlp1_body,
        grid=(NPAD // BR,),
        in_specs=[
            pl.BlockSpec((BR, DIN), lambda i: (i, 0)),
            pl.BlockSpec((NC, BR, DIN), lambda i: (0, i, 0)),
            pl.BlockSpec((DIN, DH), lambda i: (0, 0)),
            pl.BlockSpec((1, DH), lambda i: (0, 0)),
            pl.BlockSpec((DH, DH), lambda i: (0, 0)),
            pl.BlockSpec((1, DH), lambda i: (0, 0)),
        ],
        out_specs=pl.BlockSpec((NC, BR, HALF), lambda i: (0, i, 0)),
        out_shape=jax.ShapeDtypeStruct((NC, NPAD, HALF), jnp.float32),
    )(x_pad, a, W1a, b1a.reshape(1, -1), W1b, b1b.reshape(1, -1))

    h1f = h1.reshape(NC * NPAD, HALF)
    z2 = _agg2(h1f, src2, dst2)

    Wm2p = jnp.zeros((DH, HALF), jnp.float32).at[:, :Wm2.shape[1]].set(Wm2)
    bm2p = jnp.zeros((1, HALF), jnp.float32).at[0, :bm2.shape[0]].set(bm2)

    out = pl.pallas_call(
        _mlp2_body,
        grid=(NPAD // BR,),
        in_specs=[
            pl.BlockSpec((NC, BR, HALF), lambda i: (0, i, 0)),
            pl.BlockSpec((DH, DH), lambda i: (0, 0)),
            pl.BlockSpec((1, DH), lambda i: (0, 0)),
            pl.BlockSpec((DH, DH), lambda i: (0, 0)),
            pl.BlockSpec((1, DH), lambda i: (0, 0)),
            pl.BlockSpec((DH, DH), lambda i: (0, 0)),
            pl.BlockSpec((1, DH), lambda i: (0, 0)),
            pl.BlockSpec((DH, HALF), lambda i: (0, 0)),
            pl.BlockSpec((1, HALF), lambda i: (0, 0)),
        ],
        out_specs=pl.BlockSpec((BR, HALF), lambda i: (i, 0)),
        out_shape=jax.ShapeDtypeStruct((NPAD, HALF), jnp.float32),
    )(z2, W2a, b2a.reshape(1, -1), W2b, b2b.reshape(1, -1),
      Wm1, bm1.reshape(1, -1), Wm2p, bm2p)

    return out[:N, :Wm2.shape[1]]


# speedup vs baseline: 3.3305x; 1.0805x over previous
import jax
import jax.numpy as jnp
from jax import lax
from jax.experimental import pallas as pl
from jax.experimental.pallas import tpu as pltpu
from jax.experimental.pallas import tpu_sc as plsc

N = 10000
NPAD = 10240
E = 320000
EPAD = 327680
PADE = EPAD - E
DIN = 128
DH = 256
HALF = 128
NC = 2
NS = 16
ROWS_PER_SUB = NPAD // NS
CHUNK = 128
NCH1 = EPAD // (NC * NS * CHUNK)
NCH2 = EPAD // (NS * CHUNK)
IBLK = 8
RB = 128
BR = 512

_MESH = plsc.VectorSubcoreMesh(core_axis_name="c", subcore_axis_name="s")


def _zero_rows(rows):
    @pl.loop(0, RB)
    def _(i):
        for j in range(HALF // 16):
            rows[i, pl.ds(j * 16, 16)] = jnp.zeros((16,), jnp.float32)


def _edge_block(tbl_hbm, accum, sidx, didx, rows, semg, sems):
    g = {}
    s = {}
    g[0] = pltpu.async_copy(tbl_hbm.at[sidx.at[0]], rows[0], semg[0])
    for j in range(IBLK):
        g[j].wait()
        s[j] = pltpu.async_copy(rows[j % 2], accum.at[didx.at[j]],
                                sems[j % 2], add=True)
        if j >= 1:
            s[j - 1].wait()
        if j + 1 < IBLK:
            g[j + 1] = pltpu.async_copy(tbl_hbm.at[sidx.at[j + 1]],
                                        rows[(j + 1) % 2], semg[(j + 1) % 2])
    s[IBLK - 1].wait()


def _agg1_body(x_hbm, src_hbm, dst_hbm, out_hbm,
               sidx, didx, rows0, rows1, accum,
               semg0, semg1, sems0, sems1):
    c = lax.axis_index("c")
    sid = lax.axis_index("s")
    _zero_rows(rows0)

    @pl.loop(0, ROWS_PER_SUB // RB)
    def _(k):
        pltpu.sync_copy(rows0, accum.at[pl.ds(sid * ROWS_PER_SUB + k * RB, RB)])

    plsc.subcore_barrier()

    @pl.loop(0, NCH1 // IBLK)
    def _(b):
        pltpu.sync_copy(src_hbm.at[c, sid, pl.ds(b * IBLK, IBLK)], sidx)
        pltpu.sync_copy(dst_hbm.at[c, sid, pl.ds(b * IBLK, IBLK)], didx)
        _edge_block(x_hbm, accum, sidx, didx,
                    (rows0, rows1), (semg0, semg1), (sems0, sems1))

    plsc.subcore_barrier()

    @pl.loop(0, ROWS_PER_SUB // RB)
    def _(k):
        base = sid * ROWS_PER_SUB + k * RB
        pltpu.sync_copy(accum.at[pl.ds(base, RB)], rows0)
        pltpu.sync_copy(rows0, out_hbm.at[c, pl.ds(base, RB)])


def _agg2_body(h_hbm, src_hbm, dst_hbm, out_hbm,
               sidx, didx, rows0, rows1, accum,
               semg0, semg1, sems0, sems1):
    c = lax.axis_index("c")
    sid = lax.axis_index("s")

    @pl.loop(0, ROWS_PER_SUB // RB)
    def _(k):
        base = sid * ROWS_PER_SUB + k * RB
        pltpu.sync_copy(h_hbm.at[pl.ds(c * NPAD + base, RB)], rows0)
        pltpu.sync_copy(rows0, accum.at[pl.ds(base, RB)])

    plsc.subcore_barrier()

    @pl.loop(0, NCH2 // IBLK)
    def _(b):
        pltpu.sync_copy(src_hbm.at[c, sid, pl.ds(b * IBLK, IBLK)], sidx)
        pltpu.sync_copy(dst_hbm.at[sid, pl.ds(b * IBLK, IBLK)], didx)
        _edge_block(h_hbm, accum, sidx, didx,
                    (rows0, rows1), (semg0, semg1), (sems0, sems1))

    plsc.subcore_barrier()

    @pl.loop(0, ROWS_PER_SUB // RB)
    def _(k):
        base = sid * ROWS_PER_SUB + k * RB
        pltpu.sync_copy(accum.at[pl.ds(base, RB)], rows0)
        pltpu.sync_copy(rows0, out_hbm.at[c, pl.ds(base, RB)])


_agg1 = pl.kernel(
    _agg1_body,
    out_type=jax.ShapeDtypeStruct((NC, NPAD, DIN), jnp.float32),
    mesh=_MESH,
    scratch_types=[
        pltpu.VMEM((IBLK, CHUNK), jnp.int32),
        pltpu.VMEM((IBLK, CHUNK), jnp.int32),
        pltpu.VMEM((CHUNK, DIN), jnp.float32),
        pltpu.VMEM((CHUNK, DIN), jnp.float32),
        pltpu.VMEM_SHARED((NPAD, DIN), jnp.float32),
        pltpu.SemaphoreType.DMA,
        pltpu.SemaphoreType.DMA,
        pltpu.SemaphoreType.DMA,
        pltpu.SemaphoreType.DMA,
    ],
)

_agg2 = pl.kernel(
    _agg2_body,
    out_type=jax.ShapeDtypeStruct((NC, NPAD, HALF), jnp.float32),
    mesh=_MESH,
    scratch_types=[
        pltpu.VMEM((IBLK, CHUNK), jnp.int32),
        pltpu.VMEM((IBLK, CHUNK), jnp.int32),
        pltpu.VMEM((CHUNK, HALF), jnp.float32),
        pltpu.VMEM((CHUNK, HALF), jnp.float32),
        pltpu.VMEM_SHARED((NPAD, HALF), jnp.float32),
        pltpu.SemaphoreType.DMA,
        pltpu.SemaphoreType.DMA,
        pltpu.SemaphoreType.DMA,
        pltpu.SemaphoreType.DMA,
    ],
)


def _mlp1_body(x_ref, a_ref, wa_ref, ba_ref, wb_ref, bb_ref, o_ref):
    z = x_ref[...] + a_ref[0] + a_ref[1]
    t = jnp.dot(z, wa_ref[...], preferred_element_type=jnp.float32) + ba_ref[...]
    t = jnp.maximum(t, 0.0)
    u = jnp.dot(t, wb_ref[...], preferred_element_type=jnp.float32) + bb_ref[...]
    u = jnp.maximum(u, 0.0)
    o_ref[0] = u[:, :HALF]
    o_ref[1] = u[:, HALF:]


def _mlp2_body(z_ref, w2a_ref, b2a_ref, w2b_ref, b2b_ref,
               wm1_ref, bm1_ref, wm2_ref, bm2_ref, o_ref):
    t = (jnp.dot(z_ref[0], w2a_ref[:HALF], preferred_element_type=jnp.float32)
         + jnp.dot(z_ref[1], w2a_ref[HALF:], preferred_element_type=jnp.float32)
         + b2a_ref[...])
    t = jnp.maximum(t, 0.0)
    v = jnp.dot(t, w2b_ref[...], preferred_element_type=jnp.float32) + b2b_ref[...]
    h2 = jnp.maximum(v, 0.0)
    m = jnp.dot(h2, wm1_ref[...], preferred_element_type=jnp.float32) + bm1_ref[...]
    m = jnp.maximum(m, 0.0)
    o_ref[...] = jnp.dot(m, wm2_ref[...], preferred_element_type=jnp.float32) + bm2_ref[...]


def kernel(x, edge_index, W1a, b1a, W1b, b1b, W2a, b2a, W2b, b2b, Wm1, bm1, Wm2, bm2):
    src = edge_index[0].astype(jnp.int32)
    dst = edge_index[1].astype(jnp.int32)
    src_p = jnp.concatenate([src, jnp.zeros((PADE,), jnp.int32)])
    dst_p = jnp.concatenate([dst, jnp.full((PADE,), NPAD - 1, jnp.int32)])

    src1 = src_p.reshape(NC, NS, NCH1, CHUNK)
    dst1 = dst_p.reshape(NC, NS, NCH1, CHUNK)
    src2 = (src_p[None, :] + (jnp.arange(NC, dtype=jnp.int32) * NPAD)[:, None]
            ).reshape(NC, NS, NCH2, CHUNK)
    dst2 = dst_p.reshape(NS, NCH2, CHUNK)

    x_pad = jnp.pad(x, ((0, NPAD - N), (0, 0)))

    a = _agg1(x, src1, dst1)

    h1 = pl.pallas_call(
        _mlp1_body,
        grid=(NPAD // BR,),
        in_specs=[
            pl.BlockSpec((BR, DIN), lambda i: (i, 0)),
            pl.BlockSpec((NC, BR, DIN), lambda i: (0, i, 0)),
            pl.BlockSpec((DIN, DH), lambda i: (0, 0)),
            pl.BlockSpec((1, DH), lambda i: (0, 0)),
            pl.BlockSpec((DH, DH), lambda i: (0, 0)),
            pl.BlockSpec((1, DH), lambda i: (0, 0)),
        ],
        out_specs=pl.BlockSpec((NC, BR, HALF), lambda i: (0, i, 0)),
        out_shape=jax.ShapeDtypeStruct((NC, NPAD, HALF), jnp.float32),
    )(x_pad, a, W1a, b1a.reshape(1, -1), W1b, b1b.reshape(1, -1))

    h1f = h1.reshape(NC * NPAD, HALF)
    z2 = _agg2(h1f, src2, dst2)

    Wm2p = jnp.zeros((DH, HALF), jnp.float32).at[:, :Wm2.shape[1]].set(Wm2)
    bm2p = jnp.zeros((1, HALF), jnp.float32).at[0, :bm2.shape[0]].set(bm2)

    out = pl.pallas_call(
        _mlp2_body,
        grid=(NPAD // BR,),
        in_specs=[
            pl.BlockSpec((NC, BR, HALF), lambda i: (0, i, 0)),
            pl.BlockSpec((DH, DH), lambda i: (0, 0)),
            pl.BlockSpec((1, DH), lambda i: (0, 0)),
            pl.BlockSpec((DH, DH), lambda i: (0, 0)),
            pl.BlockSpec((1, DH), lambda i: (0, 0)),
            pl.BlockSpec((DH, DH), lambda i: (0, 0)),
            pl.BlockSpec((1, DH), lambda i: (0, 0)),
            pl.BlockSpec((DH, HALF), lambda i: (0, 0)),
            pl.BlockSpec((1, HALF), lambda i: (0, 0)),
        ],
        out_specs=pl.BlockSpec((BR, HALF), lambda i: (i, 0)),
        out_shape=jax.ShapeDtypeStruct((NPAD, HALF), jnp.float32),
    )(z2, W2a, b2a.reshape(1, -1), W2b, b2b.reshape(1, -1),
      Wm1, bm1.reshape(1, -1), Wm2p, bm2p)

    return out[:N, :Wm2.shape[1]]


# speedup vs baseline: 8.0935x; 2.4301x over previous
import jax
import jax.numpy as jnp
from jax import lax
from jax.experimental import pallas as pl
from jax.experimental.pallas import tpu as pltpu
from jax.experimental.pallas import tpu_sc as plsc

N = 10000
NPAD = 10240
E = 320000
EPAD = 327680
PADE = EPAD - E
DIN = 128
DH = 256
HALF = 128
NC = 2
NS = 16
ROWS_PER_SUB = NPAD // NS
CHUNK = 128
NCH1 = EPAD // (NC * NS * CHUNK)
NCH2 = EPAD // (NS * CHUNK)
IBLK = 8
RB = 128
BR = 512

_MESH = plsc.VectorSubcoreMesh(core_axis_name="c", subcore_axis_name="s")


def _zero_rows(rows):
    @pl.loop(0, RB)
    def _(i):
        for j in range(HALF // 16):
            rows[i, pl.ds(j * 16, 16)] = jnp.zeros((16,), jnp.float32)


def _edge_block(tbl_hbm, accum, sidx, didx, rows, semg, sems):
    g = {}
    s = {}
    g[0] = pltpu.async_copy(tbl_hbm.at[sidx.at[0]], rows[0], semg[0])
    for j in range(IBLK):
        g[j].wait()
        s[j] = pltpu.async_copy(rows[j % 2], accum.at[didx.at[j]],
                                sems[j % 2], add=True)
        if j >= 1:
            s[j - 1].wait()
        if j + 1 < IBLK:
            g[j + 1] = pltpu.async_copy(tbl_hbm.at[sidx.at[j + 1]],
                                        rows[(j + 1) % 2], semg[(j + 1) % 2])
    s[IBLK - 1].wait()


def _agg1_body(x_hbm, src_hbm, dst_hbm, out_hbm,
               sidx, didx, rows0, rows1, accum,
               semg0, semg1, sems0, sems1):
    c = lax.axis_index("c")
    sid = lax.axis_index("s")
    _zero_rows(rows0)

    @pl.loop(0, ROWS_PER_SUB // RB)
    def _(k):
        pltpu.sync_copy(rows0, accum.at[pl.ds(sid * ROWS_PER_SUB + k * RB, RB)])

    plsc.subcore_barrier()

    @pl.loop(0, NCH1 // IBLK)
    def _(b):
        pltpu.sync_copy(src_hbm.at[c, sid, pl.ds(b * IBLK, IBLK)], sidx)
        pltpu.sync_copy(dst_hbm.at[c, sid, pl.ds(b * IBLK, IBLK)], didx)
        _edge_block(x_hbm, accum, sidx, didx,
                    (rows0, rows1), (semg0, semg1), (sems0, sems1))

    plsc.subcore_barrier()

    @pl.loop(0, ROWS_PER_SUB // RB)
    def _(k):
        base = sid * ROWS_PER_SUB + k * RB
        pltpu.sync_copy(accum.at[pl.ds(base, RB)], rows0)
        pltpu.sync_copy(rows0, out_hbm.at[c, pl.ds(base, RB)])


def _agg2_body(h_hbm, src_hbm, dst_hbm, out_hbm,
               sidx, didx, rows0, rows1, accum,
               semg0, semg1, sems0, sems1):
    c = lax.axis_index("c")
    sid = lax.axis_index("s")

    @pl.loop(0, ROWS_PER_SUB // RB)
    def _(k):
        base = sid * ROWS_PER_SUB + k * RB
        pltpu.sync_copy(h_hbm.at[pl.ds(c * NPAD + base, RB)], rows0)
        pltpu.sync_copy(rows0, accum.at[pl.ds(base, RB)])

    plsc.subcore_barrier()

    @pl.loop(0, NCH2 // IBLK)
    def _(b):
        pltpu.sync_copy(src_hbm.at[c, sid, pl.ds(b * IBLK, IBLK)], sidx)
        pltpu.sync_copy(dst_hbm.at[sid, pl.ds(b * IBLK, IBLK)], didx)
        _edge_block(h_hbm, accum, sidx, didx,
                    (rows0, rows1), (semg0, semg1), (sems0, sems1))

    plsc.subcore_barrier()

    @pl.loop(0, ROWS_PER_SUB // RB)
    def _(k):
        base = sid * ROWS_PER_SUB + k * RB
        pltpu.sync_copy(accum.at[pl.ds(base, RB)], rows0)
        pltpu.sync_copy(rows0, out_hbm.at[c, pl.ds(base, RB)])


_agg1 = pl.kernel(
    _agg1_body,
    out_type=jax.ShapeDtypeStruct((NC, NPAD, DIN), jnp.float32),
    mesh=_MESH,
    scratch_types=[
        pltpu.VMEM((IBLK, CHUNK), jnp.int32),
        pltpu.VMEM((IBLK, CHUNK), jnp.int32),
        pltpu.VMEM((CHUNK, DIN), jnp.float32),
        pltpu.VMEM((CHUNK, DIN), jnp.float32),
        pltpu.VMEM_SHARED((NPAD, DIN), jnp.float32),
        pltpu.SemaphoreType.DMA,
        pltpu.SemaphoreType.DMA,
        pltpu.SemaphoreType.DMA,
        pltpu.SemaphoreType.DMA,
    ],
)

_agg2 = pl.kernel(
    _agg2_body,
    out_type=jax.ShapeDtypeStruct((NC, NPAD, HALF), jnp.float32),
    mesh=_MESH,
    scratch_types=[
        pltpu.VMEM((IBLK, CHUNK), jnp.int32),
        pltpu.VMEM((IBLK, CHUNK), jnp.int32),
        pltpu.VMEM((CHUNK, HALF), jnp.float32),
        pltpu.VMEM((CHUNK, HALF), jnp.float32),
        pltpu.VMEM_SHARED((NPAD, HALF), jnp.float32),
        pltpu.SemaphoreType.DMA,
        pltpu.SemaphoreType.DMA,
        pltpu.SemaphoreType.DMA,
        pltpu.SemaphoreType.DMA,
    ],
)


def _mlp1_body(x_ref, a_ref, wa_ref, ba_ref, wb_ref, bb_ref, o_ref):
    z = x_ref[...] + a_ref[0] + a_ref[1]
    t = jnp.dot(z, wa_ref[...], preferred_element_type=jnp.float32) + ba_ref[...]
    t = jnp.maximum(t, 0.0)
    u = jnp.dot(t, wb_ref[...], preferred_element_type=jnp.float32) + bb_ref[...]
    u = jnp.maximum(u, 0.0)
    o_ref[0] = u[:, :HALF]
    o_ref[1] = u[:, HALF:]


def _mlp2_body(z_ref, w2a_ref, b2a_ref, w2b_ref, b2b_ref,
               wm1_ref, bm1_ref, wm2_ref, bm2_ref, o_ref):
    t = (jnp.dot(z_ref[0], w2a_ref[:HALF], preferred_element_type=jnp.float32)
         + jnp.dot(z_ref[1], w2a_ref[HALF:], preferred_element_type=jnp.float32)
         + b2a_ref[...])
    t = jnp.maximum(t, 0.0)
    v = jnp.dot(t, w2b_ref[...], preferred_element_type=jnp.float32) + b2b_ref[...]
    h2 = jnp.maximum(v, 0.0)
    m = jnp.dot(h2, wm1_ref[...], preferred_element_type=jnp.float32) + bm1_ref[...]
    m = jnp.maximum(m, 0.0)
    o_ref[...] = jnp.dot(m, wm2_ref[...], preferred_element_type=jnp.float32) + bm2_ref[...]


def kernel(x, edge_index, W1a, b1a, W1b, b1b, W2a, b2a, W2b, b2b, Wm1, bm1, Wm2, bm2):
    src = edge_index[0].astype(jnp.int32)
    dst = edge_index[1].astype(jnp.int32)
    pad_iota = jnp.arange(PADE, dtype=jnp.int32)
    src_p = jnp.concatenate([src, pad_iota % N])
    dst_p = jnp.concatenate([dst, N + pad_iota % (NPAD - N)])

    src1 = src_p.reshape(NC, NS, NCH1, CHUNK)
    dst1 = dst_p.reshape(NC, NS, NCH1, CHUNK)
    src2 = (src_p[None, :] + (jnp.arange(NC, dtype=jnp.int32) * NPAD)[:, None]
            ).reshape(NC, NS, NCH2, CHUNK)
    dst2 = dst_p.reshape(NS, NCH2, CHUNK)

    x_pad = jnp.pad(x, ((0, NPAD - N), (0, 0)))

    a = _agg1(x, src1, dst1)

    h1 = pl.pallas_call(
        _mlp1_body,
        grid=(NPAD // BR,),
        in_specs=[
            pl.BlockSpec((BR, DIN), lambda i: (i, 0)),
            pl.BlockSpec((NC, BR, DIN), lambda i: (0, i, 0)),
            pl.BlockSpec((DIN, DH), lambda i: (0, 0)),
            pl.BlockSpec((1, DH), lambda i: (0, 0)),
            pl.BlockSpec((DH, DH), lambda i: (0, 0)),
            pl.BlockSpec((1, DH), lambda i: (0, 0)),
        ],
        out_specs=pl.BlockSpec((NC, BR, HALF), lambda i: (0, i, 0)),
        out_shape=jax.ShapeDtypeStruct((NC, NPAD, HALF), jnp.float32),
    )(x_pad, a, W1a, b1a.reshape(1, -1), W1b, b1b.reshape(1, -1))

    h1f = h1.reshape(NC * NPAD, HALF)
    z2 = _agg2(h1f, src2, dst2)

    Wm2p = jnp.zeros((DH, HALF), jnp.float32).at[:, :Wm2.shape[1]].set(Wm2)
    bm2p = jnp.zeros((1, HALF), jnp.float32).at[0, :bm2.shape[0]].set(bm2)

    out = pl.pallas_call(
        _mlp2_body,
        grid=(NPAD // BR,),
        in_specs=[
            pl.BlockSpec((NC, BR, HALF), lambda i: (0, i, 0)),
            pl.BlockSpec((DH, DH), lambda i: (0, 0)),
            pl.BlockSpec((1, DH), lambda i: (0, 0)),
            pl.BlockSpec((DH, DH), lambda i: (0, 0)),
            pl.BlockSpec((1, DH), lambda i: (0, 0)),
            pl.BlockSpec((DH, DH), lambda i: (0, 0)),
            pl.BlockSpec((1, DH), lambda i: (0, 0)),
            pl.BlockSpec((DH, HALF), lambda i: (0, 0)),
            pl.BlockSpec((1, HALF), lambda i: (0, 0)),
        ],
        out_specs=pl.BlockSpec((BR, HALF), lambda i: (i, 0)),
        out_shape=jax.ShapeDtypeStruct((NPAD, HALF), jnp.float32),
    )(z2, W2a, b2a.reshape(1, -1), W2b, b2b.reshape(1, -1),
      Wm1, bm1.reshape(1, -1), Wm2p, bm2p)

    return out[:N, :Wm2.shape[1]]


# speedup vs baseline: 8.5941x; 1.0619x over previous
import jax
import jax.numpy as jnp
from jax import lax
from jax.experimental import pallas as pl
from jax.experimental.pallas import tpu as pltpu
from jax.experimental.pallas import tpu_sc as plsc

N = 10000
NPAD = 10240
E = 320000
EPAD = 327680
PADE = EPAD - E
DIN = 128
DH = 256
HALF = 128
NC = 2
NS = 16
ROWS_PER_SUB = NPAD // NS
CHUNK = 128
NCH1 = EPAD // (NC * NS * CHUNK)
NCH2 = EPAD // (NS * CHUNK)
IBLK = 16
RB = 128
BR = 512

_MESH = plsc.VectorSubcoreMesh(core_axis_name="c", subcore_axis_name="s")


def _zero_rows(rows):
    @pl.loop(0, RB)
    def _(i):
        for j in range(HALF // 16):
            rows[i, pl.ds(j * 16, 16)] = jnp.zeros((16,), jnp.float32)


def _edge_block(tbl_hbm, accum, sidx, didx, rows, semg, sems):
    g = {}
    s = {}
    g[0] = pltpu.async_copy(tbl_hbm.at[sidx.at[0]], rows[0], semg[0])
    for j in range(IBLK):
        g[j].wait()
        s[j] = pltpu.async_copy(rows[j % 2], accum.at[didx.at[j]],
                                sems[j % 2], add=True)
        if j >= 1:
            s[j - 1].wait()
        if j + 1 < IBLK:
            g[j + 1] = pltpu.async_copy(tbl_hbm.at[sidx.at[j + 1]],
                                        rows[(j + 1) % 2], semg[(j + 1) % 2])
    s[IBLK - 1].wait()


def _agg1_body(x_hbm, src_hbm, dst_hbm, out_hbm,
               sidx, didx, rows0, rows1, accum,
               semg0, semg1, sems0, sems1):
    c = lax.axis_index("c")
    sid = lax.axis_index("s")
    _zero_rows(rows0)

    @pl.loop(0, ROWS_PER_SUB // RB)
    def _(k):
        pltpu.sync_copy(rows0, accum.at[pl.ds(sid * ROWS_PER_SUB + k * RB, RB)])

    plsc.subcore_barrier()

    @pl.loop(0, NCH1 // IBLK)
    def _(b):
        pltpu.sync_copy(src_hbm.at[c, sid, pl.ds(b * IBLK, IBLK)], sidx)
        pltpu.sync_copy(dst_hbm.at[c, sid, pl.ds(b * IBLK, IBLK)], didx)
        _edge_block(x_hbm, accum, sidx, didx,
                    (rows0, rows1), (semg0, semg1), (sems0, sems1))

    plsc.subcore_barrier()
    base0 = sid * ROWS_PER_SUB
    pltpu.sync_copy(accum.at[pl.ds(base0, ROWS_PER_SUB)],
                    out_hbm.at[c, pl.ds(base0, ROWS_PER_SUB)])


def _agg2_body(h_hbm, src_hbm, dst_hbm, out_hbm,
               sidx, didx, rows0, rows1, accum,
               semg0, semg1, sems0, sems1):
    c = lax.axis_index("c")
    sid = lax.axis_index("s")

    base0 = sid * ROWS_PER_SUB
    pltpu.sync_copy(h_hbm.at[pl.ds(c * NPAD + base0, ROWS_PER_SUB)],
                    accum.at[pl.ds(base0, ROWS_PER_SUB)])
    plsc.subcore_barrier()

    @pl.loop(0, NCH2 // IBLK)
    def _(b):
        pltpu.sync_copy(src_hbm.at[c, sid, pl.ds(b * IBLK, IBLK)], sidx)
        pltpu.sync_copy(dst_hbm.at[sid, pl.ds(b * IBLK, IBLK)], didx)
        _edge_block(h_hbm, accum, sidx, didx,
                    (rows0, rows1), (semg0, semg1), (sems0, sems1))

    plsc.subcore_barrier()
    pltpu.sync_copy(accum.at[pl.ds(base0, ROWS_PER_SUB)],
                    out_hbm.at[c, pl.ds(base0, ROWS_PER_SUB)])


_agg1 = pl.kernel(
    _agg1_body,
    out_type=jax.ShapeDtypeStruct((NC, NPAD, DIN), jnp.float32),
    mesh=_MESH,
    scratch_types=[
        pltpu.VMEM((IBLK, CHUNK), jnp.int32),
        pltpu.VMEM((IBLK, CHUNK), jnp.int32),
        pltpu.VMEM((CHUNK, DIN), jnp.float32),
        pltpu.VMEM((CHUNK, DIN), jnp.float32),
        pltpu.VMEM_SHARED((NPAD, DIN), jnp.float32),
        pltpu.SemaphoreType.DMA,
        pltpu.SemaphoreType.DMA,
        pltpu.SemaphoreType.DMA,
        pltpu.SemaphoreType.DMA,
    ],
)

_agg2 = pl.kernel(
    _agg2_body,
    out_type=jax.ShapeDtypeStruct((NC, NPAD, HALF), jnp.float32),
    mesh=_MESH,
    scratch_types=[
        pltpu.VMEM((IBLK, CHUNK), jnp.int32),
        pltpu.VMEM((IBLK, CHUNK), jnp.int32),
        pltpu.VMEM((CHUNK, HALF), jnp.float32),
        pltpu.VMEM((CHUNK, HALF), jnp.float32),
        pltpu.VMEM_SHARED((NPAD, HALF), jnp.float32),
        pltpu.SemaphoreType.DMA,
        pltpu.SemaphoreType.DMA,
        pltpu.SemaphoreType.DMA,
        pltpu.SemaphoreType.DMA,
    ],
)


def _mlp1_body(x_ref, a_ref, wa_ref, ba_ref, wb_ref, bb_ref, o_ref):
    z = x_ref[...] + a_ref[0] + a_ref[1]
    t = jnp.dot(z, wa_ref[...], preferred_element_type=jnp.float32) + ba_ref[...]
    t = jnp.maximum(t, 0.0)
    u = jnp.dot(t, wb_ref[...], preferred_element_type=jnp.float32) + bb_ref[...]
    u = jnp.maximum(u, 0.0)
    o_ref[0] = u[:, :HALF]
    o_ref[1] = u[:, HALF:]


def _mlp2_body(z_ref, w2a_ref, b2a_ref, w2b_ref, b2b_ref,
               wm1_ref, bm1_ref, wm2_ref, bm2_ref, o_ref):
    t = (jnp.dot(z_ref[0], w2a_ref[:HALF], preferred_element_type=jnp.float32)
         + jnp.dot(z_ref[1], w2a_ref[HALF:], preferred_element_type=jnp.float32)
         + b2a_ref[...])
    t = jnp.maximum(t, 0.0)
    v = jnp.dot(t, w2b_ref[...], preferred_element_type=jnp.float32) + b2b_ref[...]
    h2 = jnp.maximum(v, 0.0)
    m = jnp.dot(h2, wm1_ref[...], preferred_element_type=jnp.float32) + bm1_ref[...]
    m = jnp.maximum(m, 0.0)
    o_ref[...] = jnp.dot(m, wm2_ref[...], preferred_element_type=jnp.float32) + bm2_ref[...]


def kernel(x, edge_index, W1a, b1a, W1b, b1b, W2a, b2a, W2b, b2b, Wm1, bm1, Wm2, bm2):
    src = edge_index[0].astype(jnp.int32)
    dst = edge_index[1].astype(jnp.int32)
    pad_iota = jnp.arange(PADE, dtype=jnp.int32)
    src_p = jnp.concatenate([src, pad_iota % N])
    dst_p = jnp.concatenate([dst, N + pad_iota % (NPAD - N)])

    src1 = src_p.reshape(NC, NS, NCH1, CHUNK)
    dst1 = dst_p.reshape(NC, NS, NCH1, CHUNK)
    src2 = (src_p[None, :] + (jnp.arange(NC, dtype=jnp.int32) * NPAD)[:, None]
            ).reshape(NC, NS, NCH2, CHUNK)
    dst2 = dst_p.reshape(NS, NCH2, CHUNK)

    x_pad = jnp.pad(x, ((0, NPAD - N), (0, 0)))

    a = _agg1(x, src1, dst1)

    h1 = pl.pallas_call(
        _mlp1_body,
        grid=(NPAD // BR,),
        in_specs=[
            pl.BlockSpec((BR, DIN), lambda i: (i, 0)),
            pl.BlockSpec((NC, BR, DIN), lambda i: (0, i, 0)),
            pl.BlockSpec((DIN, DH), lambda i: (0, 0)),
            pl.BlockSpec((1, DH), lambda i: (0, 0)),
            pl.BlockSpec((DH, DH), lambda i: (0, 0)),
            pl.BlockSpec((1, DH), lambda i: (0, 0)),
        ],
        out_specs=pl.BlockSpec((NC, BR, HALF), lambda i: (0, i, 0)),
        out_shape=jax.ShapeDtypeStruct((NC, NPAD, HALF), jnp.float32),
    )(x_pad, a, W1a, b1a.reshape(1, -1), W1b, b1b.reshape(1, -1))

    h1f = h1.reshape(NC * NPAD, HALF)
    z2 = _agg2(h1f, src2, dst2)

    Wm2p = jnp.zeros((DH, HALF), jnp.float32).at[:, :Wm2.shape[1]].set(Wm2)
    bm2p = jnp.zeros((1, HALF), jnp.float32).at[0, :bm2.shape[0]].set(bm2)

    out = pl.pallas_call(
        _mlp2_body,
        grid=(NPAD // BR,),
        in_specs=[
            pl.BlockSpec((NC, BR, HALF), lambda i: (0, i, 0)),
            pl.BlockSpec((DH, DH), lambda i: (0, 0)),
            pl.BlockSpec((1, DH), lambda i: (0, 0)),
            pl.BlockSpec((DH, DH), lambda i: (0, 0)),
            pl.BlockSpec((1, DH), lambda i: (0, 0)),
            pl.BlockSpec((DH, DH), lambda i: (0, 0)),
            pl.BlockSpec((1, DH), lambda i: (0, 0)),
            pl.BlockSpec((DH, HALF), lambda i: (0, 0)),
            pl.BlockSpec((1, HALF), lambda i: (0, 0)),
        ],
        out_specs=pl.BlockSpec((BR, HALF), lambda i: (i, 0)),
        out_shape=jax.ShapeDtypeStruct((NPAD, HALF), jnp.float32),
    )(z2, W2a, b2a.reshape(1, -1), W2b, b2b.reshape(1, -1),
      Wm1, bm1.reshape(1, -1), Wm2p, bm2p)

    return out[:N, :Wm2.shape[1]]


# speedup vs baseline: 9.1540x; 1.0651x over previous
import jax
import jax.numpy as jnp
from jax import lax
from jax.experimental import pallas as pl
from jax.experimental.pallas import tpu as pltpu
from jax.experimental.pallas import tpu_sc as plsc

N = 10000
NPAD = 10240
E = 320000
EPAD = 327680
PADE = EPAD - E
DIN = 128
DH = 256
HALF = 128
NC = 2
NS = 16
ROWS_PER_SUB = NPAD // NS
CHUNK = 64
NBUF = 4
IBLK = 8
NCH1 = EPAD // (NC * NS * CHUNK)
NCH2 = EPAD // (NS * CHUNK)
NBLK1 = NCH1 // IBLK
NBLK2 = NCH2 // IBLK
BR = 512

_MESH = plsc.VectorSubcoreMesh(core_axis_name="c", subcore_axis_name="s")


def _make_agg_body(nblk, init_stride):
    def body(tbl_hbm, init_hbm, sd_hbm, out_hbm,
             rows0, rows1, rows2, rows3, ibuf, accum,
             semi, semg0, semg1, semg2, semg3, sems0, sems1, sems2, sems3):
        c = lax.axis_index("c")
        sid = lax.axis_index("s")
        rows = (rows0, rows1, rows2, rows3)
        semg = (semg0, semg1, semg2, semg3)
        sems = (sems0, sems1, sems2, sems3)

        base0 = sid * ROWS_PER_SUB
        pltpu.sync_copy(init_hbm.at[pl.ds(c * init_stride + base0, ROWS_PER_SUB)],
                        accum.at[pl.ds(base0, ROWS_PER_SUB)])
        plsc.subcore_barrier()

        pltpu.sync_copy(sd_hbm.at[c, sid, 0], ibuf.at[0])
        pltpu.async_copy(tbl_hbm.at[ibuf.at[0, 0, 0]], rows[0], semg[0])
        pltpu.async_copy(tbl_hbm.at[ibuf.at[0, 0, 1]], rows[1], semg[1])

        @pl.loop(0, nblk)
        def _(b):
            p = b % 2
            pq = (b + 1) % 2
            bnext = jnp.minimum(b + 1, nblk - 1)
            for r in range(IBLK):
                t = r % NBUF
                t2 = (r + 2) % NBUF
                pltpu.make_async_copy(tbl_hbm.at[ibuf.at[p, 0, r]],
                                      rows[t], semg[t]).wait()
                pltpu.async_copy(rows[t], accum.at[ibuf.at[p, 1, r]],
                                 sems[t], add=True)

                def _wait_s(t2=t2):
                    pltpu.make_async_copy(rows[t2],
                                          accum.at[ibuf.at[p, 1, 0]],
                                          sems[t2]).wait()
                if r >= 2:
                    _wait_s()
                else:
                    @pl.when(b >= 1)
                    def _():
                        _wait_s()

                if r == 2:
                    pltpu.async_copy(sd_hbm.at[c, sid, bnext],
                                     ibuf.at[pq], semi)
                if r == 5:
                    pltpu.make_async_copy(sd_hbm.at[c, sid, bnext],
                                          ibuf.at[pq], semi).wait()

                if r < IBLK - 2:
                    pltpu.async_copy(tbl_hbm.at[ibuf.at[p, 0, r + 2]],
                                     rows[t2], semg[t2])
                else:
                    @pl.when(b < nblk - 1)
                    def _(r=r, t2=t2):
                        pltpu.async_copy(
                            tbl_hbm.at[ibuf.at[pq, 0, r + 2 - IBLK]],
                            rows[t2], semg[t2])

        pltpu.make_async_copy(rows[2], accum.at[ibuf.at[0, 1, 0]],
                              sems[2]).wait()
        pltpu.make_async_copy(rows[3], accum.at[ibuf.at[0, 1, 0]],
                              sems[3]).wait()
        plsc.subcore_barrier()
        pltpu.sync_copy(accum.at[pl.ds(base0, ROWS_PER_SUB)],
                        out_hbm.at[c, pl.ds(base0, ROWS_PER_SUB)])
    return body


def _make_agg(nblk, init_stride):
    return pl.kernel(
        _make_agg_body(nblk, init_stride),
        out_type=jax.ShapeDtypeStruct((NC, NPAD, HALF), jnp.float32),
        mesh=_MESH,
        scratch_types=(
            [pltpu.VMEM((CHUNK, HALF), jnp.float32)] * NBUF
            + [pltpu.VMEM((2, 2, IBLK, CHUNK), jnp.int32),
               pltpu.VMEM_SHARED((NPAD, HALF), jnp.float32)]
            + [pltpu.SemaphoreType.DMA] * 9
        ),
    )


_agg1 = _make_agg(NBLK1, 0)
_agg2 = _make_agg(NBLK2, NPAD)


def _mlp1_body(x_ref, a_ref, wa_ref, ba_ref, wb_ref, bb_ref, o_ref):
    z = x_ref[...] + a_ref[0] + a_ref[1]
    t = jnp.dot(z, wa_ref[...], preferred_element_type=jnp.float32) + ba_ref[...]
    t = jnp.maximum(t, 0.0)
    u = jnp.dot(t, wb_ref[...], preferred_element_type=jnp.float32) + bb_ref[...]
    u = jnp.maximum(u, 0.0)
    o_ref[0] = u[:, :HALF]
    o_ref[1] = u[:, HALF:]


def _mlp2_body(z_ref, w2a_ref, b2a_ref, w2b_ref, b2b_ref,
               wm1_ref, bm1_ref, wm2_ref, bm2_ref, o_ref):
    t = (jnp.dot(z_ref[0], w2a_ref[:HALF], preferred_element_type=jnp.float32)
         + jnp.dot(z_ref[1], w2a_ref[HALF:], preferred_element_type=jnp.float32)
         + b2a_ref[...])
    t = jnp.maximum(t, 0.0)
    v = jnp.dot(t, w2b_ref[...], preferred_element_type=jnp.float32) + b2b_ref[...]
    h2 = jnp.maximum(v, 0.0)
    m = jnp.dot(h2, wm1_ref[...], preferred_element_type=jnp.float32) + bm1_ref[...]
    m = jnp.maximum(m, 0.0)
    o_ref[...] = jnp.dot(m, wm2_ref[...], preferred_element_type=jnp.float32) + bm2_ref[...]


def kernel(x, edge_index, W1a, b1a, W1b, b1b, W2a, b2a, W2b, b2b, Wm1, bm1, Wm2, bm2):
    src = edge_index[0].astype(jnp.int32)
    dst = edge_index[1].astype(jnp.int32)
    pad_iota = jnp.arange(PADE, dtype=jnp.int32)
    src_p = jnp.concatenate([src, pad_iota % N])
    dst_p = jnp.concatenate([dst, N + pad_iota % (NPAD - N)])

    src1 = src_p.reshape(NC, NS, NBLK1, IBLK, CHUNK)
    dst1 = dst_p.reshape(NC, NS, NBLK1, IBLK, CHUNK)
    sd1 = jnp.stack([src1, dst1], axis=3)
    src2 = (src_p[None, :] + (jnp.arange(NC, dtype=jnp.int32) * NPAD)[:, None]
            ).reshape(NC, NS, NBLK2, IBLK, CHUNK)
    dst2 = jnp.broadcast_to(dst_p.reshape(1, NS, NBLK2, IBLK, CHUNK),
                            (NC, NS, NBLK2, IBLK, CHUNK))
    sd2 = jnp.stack([src2, dst2], axis=3)

    x_pad = jnp.pad(x, ((0, NPAD - N), (0, 0)))
    zinit = jnp.zeros((NPAD, HALF), jnp.float32)

    a = _agg1(x, zinit, sd1)

    h1 = pl.pallas_call(
        _mlp1_body,
        grid=(NPAD // BR,),
        in_specs=[
            pl.BlockSpec((BR, DIN), lambda i: (i, 0)),
            pl.BlockSpec((NC, BR, DIN), lambda i: (0, i, 0)),
            pl.BlockSpec((DIN, DH), lambda i: (0, 0)),
            pl.BlockSpec((1, DH), lambda i: (0, 0)),
            pl.BlockSpec((DH, DH), lambda i: (0, 0)),
            pl.BlockSpec((1, DH), lambda i: (0, 0)),
        ],
        out_specs=pl.BlockSpec((NC, BR, HALF), lambda i: (0, i, 0)),
        out_shape=jax.ShapeDtypeStruct((NC, NPAD, HALF), jnp.float32),
    )(x_pad, a, W1a, b1a.reshape(1, -1), W1b, b1b.reshape(1, -1))

    h1f = h1.reshape(NC * NPAD, HALF)
    z2 = _agg2(h1f, h1f, sd2)

    Wm2p = jnp.zeros((DH, HALF), jnp.float32).at[:, :Wm2.shape[1]].set(Wm2)
    bm2p = jnp.zeros((1, HALF), jnp.float32).at[0, :bm2.shape[0]].set(bm2)

    out = pl.pallas_call(
        _mlp2_body,
        grid=(NPAD // BR,),
        in_specs=[
            pl.BlockSpec((NC, BR, HALF), lambda i: (0, i, 0)),
            pl.BlockSpec((DH, DH), lambda i: (0, 0)),
            pl.BlockSpec((1, DH), lambda i: (0, 0)),
            pl.BlockSpec((DH, DH), lambda i: (0, 0)),
            pl.BlockSpec((1, DH), lambda i: (0, 0)),
            pl.BlockSpec((DH, DH), lambda i: (0, 0)),
            pl.BlockSpec((1, DH), lambda i: (0, 0)),
            pl.BlockSpec((DH, HALF), lambda i: (0, 0)),
            pl.BlockSpec((1, HALF), lambda i: (0, 0)),
        ],
        out_specs=pl.BlockSpec((BR, HALF), lambda i: (i, 0)),
        out_shape=jax.ShapeDtypeStruct((NPAD, HALF), jnp.float32),
    )(z2, W2a, b2a.reshape(1, -1), W2b, b2b.reshape(1, -1),
      Wm1, bm1.reshape(1, -1), Wm2p, bm2p)

    return out[:N, :Wm2.shape[1]]


# speedup vs baseline: 9.5198x; 1.0400x over previous
import jax
import jax.numpy as jnp
from jax import lax
from jax.experimental import pallas as pl
from jax.experimental.pallas import tpu as pltpu
from jax.experimental.pallas import tpu_sc as plsc

N = 10000
NPAD = 10240
E = 320000
EPAD = 327680
PADE = EPAD - E
DIN = 128
DH = 256
HALF = 128
NC = 2
NS = 16
ROWS_PER_SUB = NPAD // NS
CHUNK = 80
NBUF = 4
IBLK = 8
NCH1 = EPAD // (NC * NS * CHUNK)
NCH2 = EPAD // (NS * CHUNK)
NBLK1 = NCH1 // IBLK
NBLK2 = NCH2 // IBLK
BR = 512

_MESH = plsc.VectorSubcoreMesh(core_axis_name="c", subcore_axis_name="s")


def _make_agg_body(nblk, init_stride):
    def body(tbl_hbm, init_hbm, sd_hbm, out_hbm,
             rows0, rows1, rows2, rows3, ibuf, accum,
             semi, semg0, semg1, semg2, semg3, sems0, sems1, sems2, sems3):
        c = lax.axis_index("c")
        sid = lax.axis_index("s")
        rows = (rows0, rows1, rows2, rows3)
        semg = (semg0, semg1, semg2, semg3)
        sems = (sems0, sems1, sems2, sems3)

        base0 = sid * ROWS_PER_SUB
        pltpu.sync_copy(init_hbm.at[pl.ds(c * init_stride + base0, ROWS_PER_SUB)],
                        accum.at[pl.ds(base0, ROWS_PER_SUB)])
        plsc.subcore_barrier()

        pltpu.sync_copy(sd_hbm.at[c, sid, 0], ibuf.at[0])
        pltpu.async_copy(tbl_hbm.at[ibuf.at[0, 0, 0]], rows[0], semg[0])
        pltpu.async_copy(tbl_hbm.at[ibuf.at[0, 0, 1]], rows[1], semg[1])

        @pl.loop(0, nblk)
        def _(b):
            p = b % 2
            pq = (b + 1) % 2
            bnext = jnp.minimum(b + 1, nblk - 1)
            for r in range(IBLK):
                t = r % NBUF
                t2 = (r + 2) % NBUF
                pltpu.make_async_copy(tbl_hbm.at[ibuf.at[p, 0, r]],
                                      rows[t], semg[t]).wait()
                pltpu.async_copy(rows[t], accum.at[ibuf.at[p, 1, r]],
                                 sems[t], add=True)

                def _wait_s(t2=t2):
                    pltpu.make_async_copy(rows[t2],
                                          accum.at[ibuf.at[p, 1, 0]],
                                          sems[t2]).wait()
                if r >= 2:
                    _wait_s()
                else:
                    @pl.when(b >= 1)
                    def _():
                        _wait_s()

                if r == 2:
                    pltpu.async_copy(sd_hbm.at[c, sid, bnext],
                                     ibuf.at[pq], semi)
                if r == 5:
                    pltpu.make_async_copy(sd_hbm.at[c, sid, bnext],
                                          ibuf.at[pq], semi).wait()

                if r < IBLK - 2:
                    pltpu.async_copy(tbl_hbm.at[ibuf.at[p, 0, r + 2]],
                                     rows[t2], semg[t2])
                else:
                    @pl.when(b < nblk - 1)
                    def _(r=r, t2=t2):
                        pltpu.async_copy(
                            tbl_hbm.at[ibuf.at[pq, 0, r + 2 - IBLK]],
                            rows[t2], semg[t2])

        pltpu.make_async_copy(rows[2], accum.at[ibuf.at[0, 1, 0]],
                              sems[2]).wait()
        pltpu.make_async_copy(rows[3], accum.at[ibuf.at[0, 1, 0]],
                              sems[3]).wait()
        plsc.subcore_barrier()
        pltpu.sync_copy(accum.at[pl.ds(base0, ROWS_PER_SUB)],
                        out_hbm.at[c, pl.ds(base0, ROWS_PER_SUB)])
    return body


def _make_agg(nblk, init_stride):
    return pl.kernel(
        _make_agg_body(nblk, init_stride),
        out_type=jax.ShapeDtypeStruct((NC, NPAD, HALF), jnp.float32),
        mesh=_MESH,
        scratch_types=(
            [pltpu.VMEM((CHUNK, HALF), jnp.float32)] * NBUF
            + [pltpu.VMEM((2, 2, IBLK, CHUNK), jnp.int32),
               pltpu.VMEM_SHARED((NPAD, HALF), jnp.float32)]
            + [pltpu.SemaphoreType.DMA] * 9
        ),
    )


_agg1 = _make_agg(NBLK1, 0)
_agg2 = _make_agg(NBLK2, NPAD)


def _mlp1_body(x_ref, a_ref, wa_ref, ba_ref, wb_ref, bb_ref, o_ref):
    z = x_ref[...] + a_ref[0] + a_ref[1]
    t = jnp.dot(z, wa_ref[...], preferred_element_type=jnp.float32) + ba_ref[...]
    t = jnp.maximum(t, 0.0)
    u = jnp.dot(t, wb_ref[...], preferred_element_type=jnp.float32) + bb_ref[...]
    u = jnp.maximum(u, 0.0)
    o_ref[0] = u[:, :HALF]
    o_ref[1] = u[:, HALF:]


def _mlp2_body(z_ref, w2a_ref, b2a_ref, w2b_ref, b2b_ref,
               wm1_ref, bm1_ref, wm2_ref, bm2_ref, o_ref):
    t = (jnp.dot(z_ref[0], w2a_ref[:HALF], preferred_element_type=jnp.float32)
         + jnp.dot(z_ref[1], w2a_ref[HALF:], preferred_element_type=jnp.float32)
         + b2a_ref[...])
    t = jnp.maximum(t, 0.0)
    v = jnp.dot(t, w2b_ref[...], preferred_element_type=jnp.float32) + b2b_ref[...]
    h2 = jnp.maximum(v, 0.0)
    m = jnp.dot(h2, wm1_ref[...], preferred_element_type=jnp.float32) + bm1_ref[...]
    m = jnp.maximum(m, 0.0)
    o_ref[...] = jnp.dot(m, wm2_ref[...], preferred_element_type=jnp.float32) + bm2_ref[...]


def kernel(x, edge_index, W1a, b1a, W1b, b1b, W2a, b2a, W2b, b2b, Wm1, bm1, Wm2, bm2):
    src = edge_index[0].astype(jnp.int32)
    dst = edge_index[1].astype(jnp.int32)
    pad_iota = jnp.arange(PADE, dtype=jnp.int32)
    src_p = jnp.concatenate([src, pad_iota % N])
    dst_p = jnp.concatenate([dst, N + pad_iota % (NPAD - N)])

    src1 = src_p.reshape(NC, NS, NBLK1, IBLK, CHUNK)
    dst1 = dst_p.reshape(NC, NS, NBLK1, IBLK, CHUNK)
    sd1 = jnp.stack([src1, dst1], axis=3)
    src2 = (src_p[None, :] + (jnp.arange(NC, dtype=jnp.int32) * NPAD)[:, None]
            ).reshape(NC, NS, NBLK2, IBLK, CHUNK)
    dst2 = jnp.broadcast_to(dst_p.reshape(1, NS, NBLK2, IBLK, CHUNK),
                            (NC, NS, NBLK2, IBLK, CHUNK))
    sd2 = jnp.stack([src2, dst2], axis=3)

    x_pad = jnp.pad(x, ((0, NPAD - N), (0, 0)))
    zinit = jnp.zeros((NPAD, HALF), jnp.float32)

    a = _agg1(x, zinit, sd1)

    h1 = pl.pallas_call(
        _mlp1_body,
        grid=(NPAD // BR,),
        in_specs=[
            pl.BlockSpec((BR, DIN), lambda i: (i, 0)),
            pl.BlockSpec((NC, BR, DIN), lambda i: (0, i, 0)),
            pl.BlockSpec((DIN, DH), lambda i: (0, 0)),
            pl.BlockSpec((1, DH), lambda i: (0, 0)),
            pl.BlockSpec((DH, DH), lambda i: (0, 0)),
            pl.BlockSpec((1, DH), lambda i: (0, 0)),
        ],
        out_specs=pl.BlockSpec((NC, BR, HALF), lambda i: (0, i, 0)),
        out_shape=jax.ShapeDtypeStruct((NC, NPAD, HALF), jnp.float32),
    )(x_pad, a, W1a, b1a.reshape(1, -1), W1b, b1b.reshape(1, -1))

    h1f = h1.reshape(NC * NPAD, HALF)
    z2 = _agg2(h1f, h1f, sd2)

    Wm2p = jnp.zeros((DH, HALF), jnp.float32).at[:, :Wm2.shape[1]].set(Wm2)
    bm2p = jnp.zeros((1, HALF), jnp.float32).at[0, :bm2.shape[0]].set(bm2)

    out = pl.pallas_call(
        _mlp2_body,
        grid=(NPAD // BR,),
        in_specs=[
            pl.BlockSpec((NC, BR, HALF), lambda i: (0, i, 0)),
            pl.BlockSpec((DH, DH), lambda i: (0, 0)),
            pl.BlockSpec((1, DH), lambda i: (0, 0)),
            pl.BlockSpec((DH, DH), lambda i: (0, 0)),
            pl.BlockSpec((1, DH), lambda i: (0, 0)),
            pl.BlockSpec((DH, DH), lambda i: (0, 0)),
            pl.BlockSpec((1, DH), lambda i: (0, 0)),
            pl.BlockSpec((DH, HALF), lambda i: (0, 0)),
            pl.BlockSpec((1, HALF), lambda i: (0, 0)),
        ],
        out_specs=pl.BlockSpec((BR, HALF), lambda i: (i, 0)),
        out_shape=jax.ShapeDtypeStruct((NPAD, HALF), jnp.float32),
    )(z2, W2a, b2a.reshape(1, -1), W2b, b2b.reshape(1, -1),
      Wm1, bm1.reshape(1, -1), Wm2p, bm2p)

    return out[:N, :Wm2.shape[1]]


# speedup vs baseline: 9.5265x; 1.0007x over previous
import jax
import jax.numpy as jnp
from jax import lax
from jax.experimental import pallas as pl
from jax.experimental.pallas import tpu as pltpu
from jax.experimental.pallas import tpu_sc as plsc

N = 10000
NPAD = 10240
E = 320000
EPAD = 327680
PADE = EPAD - E
DIN = 128
DH = 256
HALF = 128
NC = 2
NS = 16
ROWS_PER_SUB = NPAD // NS
CHUNK = 80
NBUF = 4
IBLK = 8
NCH1 = EPAD // (NC * NS * CHUNK)
NCH2 = EPAD // (NS * CHUNK)
NBLK1 = NCH1 // IBLK
NBLK2 = NCH2 // IBLK
BR = 512

_MESH = plsc.VectorSubcoreMesh(core_axis_name="c", subcore_axis_name="s")


def _make_agg_body(nblk, init_stride):
    def body(tbl_hbm, init_hbm, sd_hbm, out_hbm,
             rows0, rows1, rows2, rows3, ibuf, accum,
             semi, semg0, semg1, semg2, semg3, sems0, sems1, sems2, sems3):
        c = lax.axis_index("c")
        sid = lax.axis_index("s")
        rows = (rows0, rows1, rows2, rows3)
        semg = (semg0, semg1, semg2, semg3)
        sems = (sems0, sems1, sems2, sems3)

        base0 = sid * ROWS_PER_SUB
        pltpu.sync_copy(init_hbm.at[pl.ds(c * init_stride + base0, ROWS_PER_SUB)],
                        accum.at[pl.ds(base0, ROWS_PER_SUB)])
        plsc.subcore_barrier()

        pltpu.sync_copy(sd_hbm.at[c, sid, 0], ibuf.at[0])
        pltpu.async_copy(tbl_hbm.at[ibuf.at[0, 0, 0]], rows[0], semg[0])
        pltpu.async_copy(tbl_hbm.at[ibuf.at[0, 0, 1]], rows[1], semg[1])

        @pl.loop(0, nblk)
        def _(b):
            p = b % 2
            pq = (b + 1) % 2
            bnext = jnp.minimum(b + 1, nblk - 1)
            for r in range(IBLK):
                t = r % NBUF
                t2 = (r + 2) % NBUF
                pltpu.make_async_copy(tbl_hbm.at[ibuf.at[p, 0, r]],
                                      rows[t], semg[t]).wait()
                pltpu.async_copy(rows[t], accum.at[ibuf.at[p, 1, r]],
                                 sems[t], add=True)

                def _wait_s(t2=t2):
                    pltpu.make_async_copy(rows[t2],
                                          accum.at[ibuf.at[p, 1, 0]],
                                          sems[t2]).wait()
                if r >= 2:
                    _wait_s()
                else:
                    @pl.when(b >= 1)
                    def _():
                        _wait_s()

                if r == 2:
                    pltpu.async_copy(sd_hbm.at[c, sid, bnext],
                                     ibuf.at[pq], semi)
                if r == 5:
                    pltpu.make_async_copy(sd_hbm.at[c, sid, bnext],
                                          ibuf.at[pq], semi).wait()

                if r < IBLK - 2:
                    pltpu.async_copy(tbl_hbm.at[ibuf.at[p, 0, r + 2]],
                                     rows[t2], semg[t2])
                else:
                    @pl.when(b < nblk - 1)
                    def _(r=r, t2=t2):
                        pltpu.async_copy(
                            tbl_hbm.at[ibuf.at[pq, 0, r + 2 - IBLK]],
                            rows[t2], semg[t2])

        pltpu.make_async_copy(rows[2], accum.at[ibuf.at[0, 1, 0]],
                              sems[2]).wait()
        pltpu.make_async_copy(rows[3], accum.at[ibuf.at[0, 1, 0]],
                              sems[3]).wait()
        plsc.subcore_barrier()
        pltpu.sync_copy(accum.at[pl.ds(base0, ROWS_PER_SUB)],
                        out_hbm.at[c, pl.ds(base0, ROWS_PER_SUB)])
    return body


def _make_agg(nblk, init_stride):
    return pl.kernel(
        _make_agg_body(nblk, init_stride),
        out_type=jax.ShapeDtypeStruct((NC, NPAD, HALF), jnp.float32),
        mesh=_MESH,
        scratch_types=(
            [pltpu.VMEM((CHUNK, HALF), jnp.float32)] * NBUF
            + [pltpu.VMEM((2, 2, IBLK, CHUNK), jnp.int32),
               pltpu.VMEM_SHARED((NPAD, HALF), jnp.float32)]
            + [pltpu.SemaphoreType.DMA] * 9
        ),
    )


_agg1 = _make_agg(NBLK1, 0)
_agg2 = _make_agg(NBLK2, NPAD)


def _bdot(a, b):
    return jnp.dot(a.astype(jnp.bfloat16), b.astype(jnp.bfloat16),
                   preferred_element_type=jnp.float32)


def _mlp1_body(x_ref, a_ref, wa_ref, ba_ref, wb_ref, bb_ref, o_ref):
    z = x_ref[...] + a_ref[0] + a_ref[1]
    t = _bdot(z, wa_ref[...]) + ba_ref[...]
    t = jnp.maximum(t, 0.0)
    u = _bdot(t, wb_ref[...]) + bb_ref[...]
    u = jnp.maximum(u, 0.0)
    o_ref[0] = u[:, :HALF]
    o_ref[1] = u[:, HALF:]


def _mlp2_body(z_ref, w2a_ref, b2a_ref, w2b_ref, b2b_ref,
               wm1_ref, bm1_ref, wm2_ref, bm2_ref, o_ref):
    t = (_bdot(z_ref[0], w2a_ref[:HALF])
         + _bdot(z_ref[1], w2a_ref[HALF:])
         + b2a_ref[...])
    t = jnp.maximum(t, 0.0)
    v = _bdot(t, w2b_ref[...]) + b2b_ref[...]
    h2 = jnp.maximum(v, 0.0)
    m = _bdot(h2, wm1_ref[...]) + bm1_ref[...]
    m = jnp.maximum(m, 0.0)
    o_ref[...] = _bdot(m, wm2_ref[...]) + bm2_ref[...]


def kernel(x, edge_index, W1a, b1a, W1b, b1b, W2a, b2a, W2b, b2b, Wm1, bm1, Wm2, bm2):
    src = edge_index[0].astype(jnp.int32)
    dst = edge_index[1].astype(jnp.int32)
    pad_iota = jnp.arange(PADE, dtype=jnp.int32)
    src_p = jnp.concatenate([src, pad_iota % N])
    dst_p = jnp.concatenate([dst, N + pad_iota % (NPAD - N)])

    src1 = src_p.reshape(NC, NS, NBLK1, IBLK, CHUNK)
    dst1 = dst_p.reshape(NC, NS, NBLK1, IBLK, CHUNK)
    sd1 = jnp.stack([src1, dst1], axis=3)
    src2 = (src_p[None, :] + (jnp.arange(NC, dtype=jnp.int32) * NPAD)[:, None]
            ).reshape(NC, NS, NBLK2, IBLK, CHUNK)
    dst2 = jnp.broadcast_to(dst_p.reshape(1, NS, NBLK2, IBLK, CHUNK),
                            (NC, NS, NBLK2, IBLK, CHUNK))
    sd2 = jnp.stack([src2, dst2], axis=3)

    x_pad = jnp.pad(x, ((0, NPAD - N), (0, 0)))
    zinit = jnp.zeros((NPAD, HALF), jnp.float32)

    a = _agg1(x, zinit, sd1)

    h1 = pl.pallas_call(
        _mlp1_body,
        grid=(NPAD // BR,),
        in_specs=[
            pl.BlockSpec((BR, DIN), lambda i: (i, 0)),
            pl.BlockSpec((NC, BR, DIN), lambda i: (0, i, 0)),
            pl.BlockSpec((DIN, DH), lambda i: (0, 0)),
            pl.BlockSpec((1, DH), lambda i: (0, 0)),
            pl.BlockSpec((DH, DH), lambda i: (0, 0)),
            pl.BlockSpec((1, DH), lambda i: (0, 0)),
        ],
        out_specs=pl.BlockSpec((NC, BR, HALF), lambda i: (0, i, 0)),
        out_shape=jax.ShapeDtypeStruct((NC, NPAD, HALF), jnp.float32),
    )(x_pad, a, W1a, b1a.reshape(1, -1), W1b, b1b.reshape(1, -1))

    h1f = h1.reshape(NC * NPAD, HALF)
    z2 = _agg2(h1f, h1f, sd2)

    Wm2p = jnp.zeros((DH, HALF), jnp.float32).at[:, :Wm2.shape[1]].set(Wm2)
    bm2p = jnp.zeros((1, HALF), jnp.float32).at[0, :bm2.shape[0]].set(bm2)

    out = pl.pallas_call(
        _mlp2_body,
        grid=(NPAD // BR,),
        in_specs=[
            pl.BlockSpec((NC, BR, HALF), lambda i: (0, i, 0)),
            pl.BlockSpec((DH, DH), lambda i: (0, 0)),
            pl.BlockSpec((1, DH), lambda i: (0, 0)),
            pl.BlockSpec((DH, DH), lambda i: (0, 0)),
            pl.BlockSpec((1, DH), lambda i: (0, 0)),
            pl.BlockSpec((DH, DH), lambda i: (0, 0)),
            pl.BlockSpec((1, DH), lambda i: (0, 0)),
            pl.BlockSpec((DH, HALF), lambda i: (0, 0)),
            pl.BlockSpec((1, HALF), lambda i: (0, 0)),
        ],
        out_specs=pl.BlockSpec((BR, HALF), lambda i: (i, 0)),
        out_shape=jax.ShapeDtypeStruct((NPAD, HALF), jnp.float32),
    )(z2, W2a, b2a.reshape(1, -1), W2b, b2b.reshape(1, -1),
      Wm1, bm1.reshape(1, -1), Wm2p, bm2p)

    return out[:N, :Wm2.shape[1]]
